# baseline (speedup 1.0000x reference)
import jax
import jax.numpy as jnp
from jax import lax
from jax.experimental import pallas as pl
from jax.experimental.pallas import tpu as pltpu
from jax.experimental.pallas import tpu_sc as plsc

NC = 2
NS = 16
NW = NC * NS
CHUNK = 128
NBUF = 8
PREF = 4


def _sc_mesh():
    return plsc.VectorSubcoreMesh(
        core_axis_name="c", subcore_axis_name="s", num_cores=NC, num_subcores=NS
    )


def _qrsqrt(x):
    i = plsc.bitcast(x, jnp.int32)
    i = jnp.int32(0x5F3759DF) - (i >> 1)
    y = plsc.bitcast(i, jnp.float32)
    xh = x * 0.5
    y = y * (1.5 - xh * y * y)
    y = y * (1.5 - xh * y * y)
    return y


def _make_deg_kernel(npad, cpt, dhid):
    rpt = npad // NS

    def body(dst_hbm, zeros_hbm, ones_hbm, out_hbm, idx_v, ones_v, ssem, sh):
        cid = lax.axis_index("c")
        sid = lax.axis_index("s")
        wid = cid * NS + sid
        pltpu.sync_copy(zeros_hbm.at[pl.ds(sid * rpt, rpt)],
                        sh.at[pl.ds(sid * rpt, rpt)])
        pltpu.sync_copy(dst_hbm.at[wid], idx_v)
        pltpu.sync_copy(ones_hbm, ones_v)
        plsc.subcore_barrier()
        descs = [None] * cpt
        for j in range(cpt):
            b = j % NBUF
            if j >= NBUF:
                descs[j - NBUF].wait()
            descs[j] = pltpu.async_copy(
                ones_v, sh.at[idx_v.at[j]], ssem.at[b], add=True)
        for j in range(max(0, cpt - NBUF), cpt):
            descs[j].wait()
        plsc.subcore_barrier()
        pltpu.sync_copy(sh.at[pl.ds(sid * rpt, rpt)],
                        out_hbm.at[cid, pl.ds(sid * rpt, rpt)])

    return pl.kernel(
        body,
        out_type=jax.ShapeDtypeStruct((NC, npad, dhid), jnp.float32),
        mesh=_sc_mesh(),
        compiler_params=pltpu.CompilerParams(use_tc_tiling_on_sc=False, needs_layout_passes=False),
        scratch_types=[
            pltpu.VMEM((cpt, CHUNK), jnp.int32),
            pltpu.VMEM((CHUNK, dhid), jnp.float32),
            pltpu.SemaphoreType.DMA((NBUF,)),
            pltpu.VMEM_SHARED((npad, dhid), jnp.float32),
        ],
    )


def _agg_pipeline(zs_sh, agg_sh, src_v, dst_v, rows_v, gsem, ssem, cpt):
    gat = [None] * cpt
    scat = [None] * cpt
    for j in range(cpt + PREF):
        if j < cpt:
            b = j % NBUF
            if j >= NBUF:
                scat[j - NBUF].wait()
            gat[j] = pltpu.async_copy(
                zs_sh.at[src_v.at[j]], rows_v.at[b], gsem.at[b])
        i = j - PREF
        if 0 <= i < cpt:
            b = i % NBUF
            gat[i].wait()
            scat[i] = pltpu.async_copy(
                rows_v.at[b], agg_sh.at[dst_v.at[i]], ssem.at[b], add=True)
    for i in range(max(0, cpt - NBUF), cpt):
        scat[i].wait()


def _make_agg1_kernel(npad, cpt, dhid):
    rpt = npad // NS

    rpt8 = rpt // 8

    def body(degp_hbm, h1_hbm, src_hbm, dst_hbm, zeros_hbm, out_hbm,
             src_v, dst_v, rows_v, p0, p1, hbuf, dinvb, zsb,
             gsem, ssem, zs_sh, agg_sh):
        cid = lax.axis_index("c")
        sid = lax.axis_index("s")
        wid = cid * NS + sid
        r0 = sid * rpt
        pltpu.sync_copy(zeros_hbm.at[pl.ds(r0, rpt)],
                        agg_sh.at[pl.ds(r0, rpt)])
        pltpu.sync_copy(degp_hbm.at[0, pl.ds(r0, rpt)], p0)
        pltpu.sync_copy(degp_hbm.at[1, pl.ds(r0, rpt)], p1)
        pltpu.sync_copy(h1_hbm.at[pl.ds(sid * rpt8, rpt8)], hbuf)
        pltpu.sync_copy(src_hbm.at[wid], src_v)
        pltpu.sync_copy(dst_hbm.at[wid], dst_v)

        @pl.loop(0, rpt8)
        def _(i):
            for j in range(8):
                k = i * 8 + j
                deg = p0[k, :] + p1[k, :] + 1.0
                dv = _qrsqrt(deg)
                dinvb[k, :] = dv
                zsb[k, :] = hbuf[i, pl.ds(j * dhid, dhid)] * dv

        pltpu.sync_copy(zsb, zs_sh.at[pl.ds(r0, rpt)])

        @pl.when(cid == 0)
        def _():
            pltpu.sync_copy(dinvb, out_hbm.at[2, pl.ds(r0, rpt)])

        plsc.subcore_barrier()
        _agg_pipeline(zs_sh, agg_sh, src_v, dst_v, rows_v, gsem, ssem, cpt)
        plsc.subcore_barrier()
        pltpu.sync_copy(agg_sh.at[pl.ds(r0, rpt)], zsb)

        @pl.loop(0, rpt)
        def _(i):
            zsb[i, :] = zsb[i, :] * dinvb[i, :]

        pltpu.sync_copy(zsb, out_hbm.at[cid, pl.ds(r0, rpt)])

    return pl.kernel(
        body,
        out_type=jax.ShapeDtypeStruct((3, npad, dhid), jnp.float32),
        mesh=_sc_mesh(),
        compiler_params=pltpu.CompilerParams(use_tc_tiling_on_sc=False, needs_layout_passes=False),
        scratch_types=[
            pltpu.VMEM((cpt, CHUNK), jnp.int32),
            pltpu.VMEM((cpt, CHUNK), jnp.int32),
            pltpu.VMEM((NBUF, CHUNK, dhid), jnp.float32),
            pltpu.VMEM((npad // NS, dhid), jnp.float32),
            pltpu.VMEM((npad // NS, dhid), jnp.float32),
            pltpu.VMEM((npad // (NS * 8), 8 * dhid), jnp.float32),
            pltpu.VMEM((npad // NS, dhid), jnp.float32),
            pltpu.VMEM((npad // NS, dhid), jnp.float32),
            pltpu.SemaphoreType.DMA((NBUF,)),
            pltpu.SemaphoreType.DMA((NBUF,)),
            pltpu.VMEM_SHARED((npad, dhid), jnp.float32),
            pltpu.VMEM_SHARED((npad, dhid), jnp.float32),
        ],
    )


def _make_agg2_kernel(npad, cpt, dhid):
    rpt = npad // NS

    rpt8 = rpt // 8

    def body(s2_hbm, h1_hbm, b1_hbm, src_hbm, dst_hbm, zeros_hbm, out_hbm,
             src_v, dst_v, rows_v, p0, p1, hbuf, dinvb, zsb, selfb, bbuf,
             gsem, ssem, zs_sh, agg_sh):
        cid = lax.axis_index("c")
        sid = lax.axis_index("s")
        wid = cid * NS + sid
        r0 = sid * rpt
        pltpu.sync_copy(zeros_hbm.at[pl.ds(r0, rpt)],
                        agg_sh.at[pl.ds(r0, rpt)])
        pltpu.sync_copy(s2_hbm.at[0, pl.ds(r0, rpt)], p0)
        pltpu.sync_copy(s2_hbm.at[1, pl.ds(r0, rpt)], p1)
        pltpu.sync_copy(s2_hbm.at[2, pl.ds(r0, rpt)], dinvb)
        pltpu.sync_copy(h1_hbm.at[pl.ds(sid * rpt8, rpt8)], hbuf)
        pltpu.sync_copy(b1_hbm, bbuf)
        pltpu.sync_copy(src_hbm.at[wid], src_v)
        pltpu.sync_copy(dst_hbm.at[wid], dst_v)

        @pl.loop(0, rpt8)
        def _(i):
            for j in range(8):
                k = i * 8 + j
                dv = dinvb[k, :]
                z = (p0[k, :] + p1[k, :]
                     + hbuf[i, pl.ds(j * dhid, dhid)] * dv * dv + bbuf[:])
                z = jnp.maximum(z, 0.0)
                zsb[k, :] = z * dv
                selfb[k, :] = z * dv * dv

        pltpu.sync_copy(zsb, zs_sh.at[pl.ds(r0, rpt)])

        @pl.when(cid == 0)
        def _():
            pltpu.sync_copy(selfb, out_hbm.at[2, pl.ds(r0, rpt)])

        plsc.subcore_barrier()
        _agg_pipeline(zs_sh, agg_sh, src_v, dst_v, rows_v, gsem, ssem, cpt)
        plsc.subcore_barrier()
        pltpu.sync_copy(agg_sh.at[pl.ds(r0, rpt)], zsb)

        @pl.loop(0, rpt)
        def _(i):
            zsb[i, :] = zsb[i, :] * dinvb[i, :]

        pltpu.sync_copy(zsb, out_hbm.at[cid, pl.ds(r0, rpt)])

    return pl.kernel(
        body,
        out_type=jax.ShapeDtypeStruct((3, npad, dhid), jnp.float32),
        mesh=_sc_mesh(),
        compiler_params=pltpu.CompilerParams(use_tc_tiling_on_sc=False, needs_layout_passes=False),
        scratch_types=[
            pltpu.VMEM((cpt, CHUNK), jnp.int32),
            pltpu.VMEM((cpt, CHUNK), jnp.int32),
            pltpu.VMEM((NBUF, CHUNK, dhid), jnp.float32),
            pltpu.VMEM((npad // NS, dhid), jnp.float32),
            pltpu.VMEM((npad // NS, dhid), jnp.float32),
            pltpu.VMEM((npad // (NS * 8), 8 * dhid), jnp.float32),
            pltpu.VMEM((npad // NS, dhid), jnp.float32),
            pltpu.VMEM((npad // NS, dhid), jnp.float32),
            pltpu.VMEM((npad // NS, dhid), jnp.float32),
            pltpu.VMEM((dhid,), jnp.float32),
            pltpu.SemaphoreType.DMA((NBUF,)),
            pltpu.SemaphoreType.DMA((NBUF,)),
            pltpu.VMEM_SHARED((npad, dhid), jnp.float32),
            pltpu.VMEM_SHARED((npad, dhid), jnp.float32),
        ],
    )


def _m1(x_ref, w_ref, h_ref):
    h_ref[...] = jnp.dot(x_ref[...], w_ref[...],
                         preferred_element_type=jnp.float32)


def _m3(q0_ref, q1_ref, q2_ref, w_ref, b_ref, out_ref):
    pre = q0_ref[...] + q1_ref[...] + q2_ref[...]
    out_ref[...] = (
        jnp.dot(pre, w_ref[...], preferred_element_type=jnp.float32) + b_ref[...]
    )


def kernel(x, edge_index, W1, b1, W2, b2):
    n, d_in = x.shape
    d_hid = W1.shape[1]
    e = edge_index.shape[1]

    cpt = -(-e // (NW * CHUNK))
    e_pad = NW * cpt * CHUNK
    npad = -(-(n + 1) // (NS * 8)) * (NS * 8)

    ep = jnp.pad(edge_index, ((0, 0), (0, e_pad - e)), constant_values=n)
    srcp = ep[0].reshape(NW, cpt, CHUNK)
    dstp = ep[1].reshape(NW, cpt, CHUNK)
    zeros_tab = jnp.zeros((npad, d_hid), jnp.float32)
    ones_blk = jnp.ones((CHUNK, d_hid), jnp.float32)

    degp = _make_deg_kernel(npad, cpt, d_hid)(dstp, zeros_tab, ones_blk)

    P = 128 // d_hid
    BR = 128
    G = -(-(n // P) // BR)
    x8 = x.reshape(n // P, P * d_in)
    w1rep = jnp.kron(jnp.eye(P, dtype=W1.dtype), W1)
    h1 = pl.pallas_call(
        _m1,
        grid=(G,),
        in_specs=[
            pl.BlockSpec((BR, P * d_in), lambda i: (i, 0)),
            pl.BlockSpec((P * d_in, P * d_hid), lambda i: (0, 0)),
        ],
        out_specs=pl.BlockSpec((BR, P * d_hid), lambda i: (i, 0)),
        out_shape=jax.ShapeDtypeStruct(
            (npad // P, P * d_hid), jnp.float32),
    )(x8, w1rep)

    s2 = _make_agg1_kernel(npad, cpt, d_hid)(degp, h1, srcp, dstp, zeros_tab)
    s3 = _make_agg2_kernel(npad, cpt, d_hid)(s2, h1, b1, srcp, dstp, zeros_tab)

    q0 = s3[0].reshape(npad // P, P * d_hid)
    q1 = s3[1].reshape(npad // P, P * d_hid)
    q2 = s3[2].reshape(npad // P, P * d_hid)
    w2rep = jnp.kron(jnp.eye(P, dtype=W2.dtype), W2)
    b2rep = jnp.tile(b2, P).reshape(1, P * d_in)
    out128 = pl.pallas_call(
        _m3,
        grid=(G,),
        in_specs=[
            pl.BlockSpec((BR, P * d_hid), lambda i: (i, 0)),
            pl.BlockSpec((BR, P * d_hid), lambda i: (i, 0)),
            pl.BlockSpec((BR, P * d_hid), lambda i: (i, 0)),
            pl.BlockSpec((P * d_hid, P * d_in), lambda i: (0, 0)),
            pl.BlockSpec((1, P * d_in), lambda i: (0, 0)),
        ],
        out_specs=pl.BlockSpec((BR, P * d_in), lambda i: (i, 0)),
        out_shape=jax.ShapeDtypeStruct((n // P, P * d_in), jnp.float32),
    )(q0, q1, q2, w2rep, b2rep)

    return out128.reshape(n, d_in)


# speedup vs baseline: 1.5282x; 1.5282x over previous
import jax
import jax.numpy as jnp
from jax import lax
from jax.experimental import pallas as pl
from jax.experimental.pallas import tpu as pltpu
from jax.experimental.pallas import tpu_sc as plsc

NC = 2
NS = 16
NW = NC * NS
CHUNK = 128
NBUF = 8
PREF = 4


def _sc_mesh():
    return plsc.VectorSubcoreMesh(
        core_axis_name="c", subcore_axis_name="s", num_cores=NC, num_subcores=NS
    )


def _qrsqrt(x):
    i = plsc.bitcast(x, jnp.int32)
    i = jnp.int32(0x5F3759DF) - (i >> 1)
    y = plsc.bitcast(i, jnp.float32)
    xh = x * 0.5
    y = y * (1.5 - xh * y * y)
    y = y * (1.5 - xh * y * y)
    return y


def _make_deg_kernel(npad, cpt, dhid):
    rpt = npad // NS

    def body(dst_hbm, zeros_hbm, ones_hbm, out_hbm, idx_v, ones_v, ssem, sh):
        cid = lax.axis_index("c")
        sid = lax.axis_index("s")
        wid = cid * NS + sid
        pltpu.sync_copy(zeros_hbm.at[pl.ds(sid * rpt, rpt)],
                        sh.at[pl.ds(sid * rpt, rpt)])
        pltpu.sync_copy(dst_hbm.at[wid], idx_v)
        pltpu.sync_copy(ones_hbm, ones_v)
        plsc.subcore_barrier()
        descs = [None] * cpt
        for j in range(cpt):
            b = j % NBUF
            if j >= NBUF:
                descs[j - NBUF].wait()
            descs[j] = pltpu.async_copy(
                ones_v, sh.at[idx_v.at[j]], ssem.at[b], add=True)
        for j in range(max(0, cpt - NBUF), cpt):
            descs[j].wait()
        plsc.subcore_barrier()
        pltpu.sync_copy(sh.at[pl.ds(sid * rpt, rpt)],
                        out_hbm.at[cid, pl.ds(sid * rpt, rpt)])

    return pl.kernel(
        body,
        out_type=jax.ShapeDtypeStruct((NC, npad, dhid), jnp.float32),
        mesh=_sc_mesh(),
        compiler_params=pltpu.CompilerParams(use_tc_tiling_on_sc=False, needs_layout_passes=False),
        scratch_types=[
            pltpu.VMEM((cpt, CHUNK), jnp.int32),
            pltpu.VMEM((CHUNK, dhid), jnp.float32),
            pltpu.SemaphoreType.DMA((NBUF,)),
            pltpu.VMEM_SHARED((npad, dhid), jnp.float32),
        ],
    )


def _agg_pipeline(zs_sh, agg_sh, src_v, dst_v, rows_v, gsem, ssem, cpt):
    gat = [None] * cpt
    scat = [None] * cpt
    for j in range(cpt + PREF):
        if j < cpt:
            b = j % NBUF
            if j >= NBUF:
                scat[j - NBUF].wait()
            gat[j] = pltpu.async_copy(
                zs_sh.at[src_v.at[j]], rows_v.at[b], gsem.at[b])
        i = j - PREF
        if 0 <= i < cpt:
            b = i % NBUF
            gat[i].wait()
            scat[i] = pltpu.async_copy(
                rows_v.at[b], agg_sh.at[dst_v.at[i]], ssem.at[b], add=True)
    for i in range(max(0, cpt - NBUF), cpt):
        scat[i].wait()


def _make_agg1_kernel(npad, cpt, dhid):
    rpt = npad // NS

    def body(degp_hbm, h1_hbm, src_hbm, dst_hbm, zeros_hbm, out_hbm,
             src_v, dst_v, rows_v, p0, p1, hbuf, dinvb, zsb,
             gsem, ssem, zs_sh, agg_sh):
        cid = lax.axis_index("c")
        sid = lax.axis_index("s")
        wid = cid * NS + sid
        r0 = sid * rpt
        pltpu.sync_copy(zeros_hbm.at[pl.ds(r0, rpt)],
                        agg_sh.at[pl.ds(r0, rpt)])
        pltpu.sync_copy(degp_hbm.at[0, pl.ds(r0, rpt)], p0)
        pltpu.sync_copy(degp_hbm.at[1, pl.ds(r0, rpt)], p1)
        pltpu.sync_copy(h1_hbm.at[pl.ds(r0, rpt)], hbuf)
        pltpu.sync_copy(src_hbm.at[wid], src_v)
        pltpu.sync_copy(dst_hbm.at[wid], dst_v)

        @pl.loop(0, rpt)
        def _(i):
            deg = p0[i, :] + p1[i, :] + 1.0
            dv = _qrsqrt(deg)
            dinvb[i, :] = dv
            zsb[i, :] = hbuf[i, :] * dv

        pltpu.sync_copy(zsb, zs_sh.at[pl.ds(r0, rpt)])

        @pl.when(cid == 0)
        def _():
            pltpu.sync_copy(dinvb, out_hbm.at[2, pl.ds(r0, rpt)])

        plsc.subcore_barrier()
        _agg_pipeline(zs_sh, agg_sh, src_v, dst_v, rows_v, gsem, ssem, cpt)
        plsc.subcore_barrier()
        pltpu.sync_copy(agg_sh.at[pl.ds(r0, rpt)], zsb)

        @pl.loop(0, rpt)
        def _(i):
            zsb[i, :] = zsb[i, :] * dinvb[i, :]

        pltpu.sync_copy(zsb, out_hbm.at[cid, pl.ds(r0, rpt)])

    return pl.kernel(
        body,
        out_type=jax.ShapeDtypeStruct((3, npad, dhid), jnp.float32),
        mesh=_sc_mesh(),
        compiler_params=pltpu.CompilerParams(use_tc_tiling_on_sc=False, needs_layout_passes=False),
        scratch_types=[
            pltpu.VMEM((cpt, CHUNK), jnp.int32),
            pltpu.VMEM((cpt, CHUNK), jnp.int32),
            pltpu.VMEM((NBUF, CHUNK, dhid), jnp.float32),
            pltpu.VMEM((npad // NS, dhid), jnp.float32),
            pltpu.VMEM((npad // NS, dhid), jnp.float32),
            pltpu.VMEM((npad // NS, dhid), jnp.float32),
            pltpu.VMEM((npad // NS, dhid), jnp.float32),
            pltpu.VMEM((npad // NS, dhid), jnp.float32),
            pltpu.SemaphoreType.DMA((NBUF,)),
            pltpu.SemaphoreType.DMA((NBUF,)),
            pltpu.VMEM_SHARED((npad, dhid), jnp.float32),
            pltpu.VMEM_SHARED((npad, dhid), jnp.float32),
        ],
    )


def _make_agg2_kernel(npad, cpt, dhid):
    rpt = npad // NS

    def body(s2_hbm, h1_hbm, b1_hbm, src_hbm, dst_hbm, zeros_hbm, out_hbm,
             src_v, dst_v, rows_v, p0, p1, hbuf, dinvb, zsb, selfb, bbuf,
             gsem, ssem, zs_sh, agg_sh):
        cid = lax.axis_index("c")
        sid = lax.axis_index("s")
        wid = cid * NS + sid
        r0 = sid * rpt
        pltpu.sync_copy(zeros_hbm.at[pl.ds(r0, rpt)],
                        agg_sh.at[pl.ds(r0, rpt)])
        pltpu.sync_copy(s2_hbm.at[0, pl.ds(r0, rpt)], p0)
        pltpu.sync_copy(s2_hbm.at[1, pl.ds(r0, rpt)], p1)
        pltpu.sync_copy(s2_hbm.at[2, pl.ds(r0, rpt)], dinvb)
        pltpu.sync_copy(h1_hbm.at[pl.ds(r0, rpt)], hbuf)
        pltpu.sync_copy(b1_hbm, bbuf)
        pltpu.sync_copy(src_hbm.at[wid], src_v)
        pltpu.sync_copy(dst_hbm.at[wid], dst_v)

        @pl.loop(0, rpt)
        def _(i):
            dv = dinvb[i, :]
            z = p0[i, :] + p1[i, :] + hbuf[i, :] * dv * dv + bbuf[:]
            z = jnp.maximum(z, 0.0)
            zsb[i, :] = z * dv
            selfb[i, :] = z * dv * dv

        pltpu.sync_copy(zsb, zs_sh.at[pl.ds(r0, rpt)])

        @pl.when(cid == 0)
        def _():
            pltpu.sync_copy(selfb, out_hbm.at[2, pl.ds(r0, rpt)])

        plsc.subcore_barrier()
        _agg_pipeline(zs_sh, agg_sh, src_v, dst_v, rows_v, gsem, ssem, cpt)
        plsc.subcore_barrier()
        pltpu.sync_copy(agg_sh.at[pl.ds(r0, rpt)], zsb)

        @pl.loop(0, rpt)
        def _(i):
            zsb[i, :] = zsb[i, :] * dinvb[i, :]

        pltpu.sync_copy(zsb, out_hbm.at[cid, pl.ds(r0, rpt)])

    return pl.kernel(
        body,
        out_type=jax.ShapeDtypeStruct((3, npad, dhid), jnp.float32),
        mesh=_sc_mesh(),
        compiler_params=pltpu.CompilerParams(use_tc_tiling_on_sc=False, needs_layout_passes=False),
        scratch_types=[
            pltpu.VMEM((cpt, CHUNK), jnp.int32),
            pltpu.VMEM((cpt, CHUNK), jnp.int32),
            pltpu.VMEM((NBUF, CHUNK, dhid), jnp.float32),
            pltpu.VMEM((npad // NS, dhid), jnp.float32),
            pltpu.VMEM((npad // NS, dhid), jnp.float32),
            pltpu.VMEM((npad // NS, dhid), jnp.float32),
            pltpu.VMEM((npad // NS, dhid), jnp.float32),
            pltpu.VMEM((npad // NS, dhid), jnp.float32),
            pltpu.VMEM((npad // NS, dhid), jnp.float32),
            pltpu.VMEM((dhid,), jnp.float32),
            pltpu.SemaphoreType.DMA((NBUF,)),
            pltpu.SemaphoreType.DMA((NBUF,)),
            pltpu.VMEM_SHARED((npad, dhid), jnp.float32),
            pltpu.VMEM_SHARED((npad, dhid), jnp.float32),
        ],
    )


def _m1(x_ref, w_ref, h_ref):
    h_ref[...] = jnp.dot(x_ref[...], w_ref[...],
                         preferred_element_type=jnp.float32)


def _m3(s_ref, w_ref, b_ref, out_ref):
    pre = s_ref[0] + s_ref[1] + s_ref[2]
    out_ref[...] = (
        jnp.dot(pre, w_ref[...], preferred_element_type=jnp.float32) + b_ref[...]
    )


def kernel(x, edge_index, W1, b1, W2, b2):
    n, d_in = x.shape
    d_hid = W1.shape[1]
    e = edge_index.shape[1]

    cpt = -(-e // (NW * CHUNK))
    e_pad = NW * cpt * CHUNK
    npad = -(-(n + 1) // (NS * 8)) * (NS * 8)

    ep = jnp.pad(edge_index, ((0, 0), (0, e_pad - e)), constant_values=n)
    srcp = ep[0].reshape(NW, cpt, CHUNK)
    dstp = ep[1].reshape(NW, cpt, CHUNK)
    zeros_tab = jnp.zeros((npad, d_hid), jnp.float32)
    ones_blk = jnp.ones((CHUNK, d_hid), jnp.float32)

    degp = _make_deg_kernel(npad, cpt, d_hid)(dstp, zeros_tab, ones_blk)

    R = 1000
    G = n // R
    h1 = pl.pallas_call(
        _m1,
        grid=(G,),
        in_specs=[
            pl.BlockSpec((R, d_in), lambda i: (i, 0)),
            pl.BlockSpec((d_in, d_hid), lambda i: (0, 0)),
        ],
        out_specs=pl.BlockSpec((R, d_hid), lambda i: (i, 0)),
        out_shape=jax.ShapeDtypeStruct((npad, d_hid), jnp.float32),
    )(x, W1)

    s2 = _make_agg1_kernel(npad, cpt, d_hid)(degp, h1, srcp, dstp, zeros_tab)
    s3 = _make_agg2_kernel(npad, cpt, d_hid)(s2, h1, b1, srcp, dstp, zeros_tab)

    out = pl.pallas_call(
        _m3,
        grid=(G,),
        in_specs=[
            pl.BlockSpec((3, R, d_hid), lambda i: (0, i, 0)),
            pl.BlockSpec((d_hid, d_in), lambda i: (0, 0)),
            pl.BlockSpec((1, d_in), lambda i: (0, 0)),
        ],
        out_specs=pl.BlockSpec((R, d_in), lambda i: (i, 0)),
        out_shape=jax.ShapeDtypeStruct((n, d_in), jnp.float32),
    )(s3, W2, b2.reshape(1, d_in))

    return out


# speedup vs baseline: 1.6324x; 1.0682x over previous
import jax
import jax.numpy as jnp
from jax import lax
from jax.experimental import pallas as pl
from jax.experimental.pallas import tpu as pltpu
from jax.experimental.pallas import tpu_sc as plsc

NC = 2
NS = 16
NW = NC * NS
CHUNK = 128
NBUF = 12
PREF = 6


def _sc_mesh():
    return plsc.VectorSubcoreMesh(
        core_axis_name="c", subcore_axis_name="s", num_cores=NC, num_subcores=NS
    )


def _qrsqrt(x):
    i = plsc.bitcast(x, jnp.int32)
    i = jnp.int32(0x5F3759DF) - (i >> 1)
    y = plsc.bitcast(i, jnp.float32)
    xh = x * 0.5
    y = y * (1.5 - xh * y * y)
    y = y * (1.5 - xh * y * y)
    return y


def _make_deg_kernel(npad, cpt, dhid):
    rpt = npad // NS

    def body(dst_hbm, zeros_hbm, ones_hbm, out_hbm, idx_v, ones_v, ssem, sh):
        cid = lax.axis_index("c")
        sid = lax.axis_index("s")
        wid = cid * NS + sid
        pltpu.sync_copy(zeros_hbm.at[pl.ds(sid * rpt, rpt)],
                        sh.at[pl.ds(sid * rpt, rpt)])
        pltpu.sync_copy(dst_hbm.at[wid], idx_v)
        pltpu.sync_copy(ones_hbm, ones_v)
        plsc.subcore_barrier()
        descs = [None] * cpt
        for j in range(cpt):
            b = j % NBUF
            if j >= NBUF:
                descs[j - NBUF].wait()
            descs[j] = pltpu.async_copy(
                ones_v, sh.at[idx_v.at[j]], ssem.at[b], add=True)
        for j in range(max(0, cpt - NBUF), cpt):
            descs[j].wait()
        plsc.subcore_barrier()
        pltpu.sync_copy(sh.at[pl.ds(sid * rpt, rpt)],
                        out_hbm.at[cid, pl.ds(sid * rpt, rpt)])

    return pl.kernel(
        body,
        out_type=jax.ShapeDtypeStruct((NC, npad, dhid), jnp.float32),
        mesh=_sc_mesh(),
        compiler_params=pltpu.CompilerParams(use_tc_tiling_on_sc=False, needs_layout_passes=False),
        scratch_types=[
            pltpu.VMEM((cpt, CHUNK), jnp.int32),
            pltpu.VMEM((CHUNK, dhid), jnp.float32),
            pltpu.SemaphoreType.DMA((NBUF,)),
            pltpu.VMEM_SHARED((npad, dhid), jnp.float32),
        ],
    )


def _agg_pipeline(zs_sh, agg_sh, src_v, dst_v, rows_v, gsem, ssem, cpt):
    gat = [None] * cpt
    scat = [None] * cpt
    for j in range(cpt + PREF):
        if j < cpt:
            b = j % NBUF
            if j >= NBUF:
                scat[j - NBUF].wait()
            gat[j] = pltpu.async_copy(
                zs_sh.at[src_v.at[j]], rows_v.at[b], gsem.at[b])
        i = j - PREF
        if 0 <= i < cpt:
            b = i % NBUF
            gat[i].wait()
            scat[i] = pltpu.async_copy(
                rows_v.at[b], agg_sh.at[dst_v.at[i]], ssem.at[b], add=True)
    for i in range(max(0, cpt - NBUF), cpt):
        scat[i].wait()


def _make_agg1_kernel(npad, cpt, dhid):
    rpt = npad // NS

    def body(degp_hbm, h1_hbm, src_hbm, dst_hbm, zeros_hbm, out_hbm,
             src_v, dst_v, rows_v, p0, p1, hbuf, dinvb, zsb,
             gsem, ssem, zs_sh, agg_sh):
        cid = lax.axis_index("c")
        sid = lax.axis_index("s")
        wid = cid * NS + sid
        r0 = sid * rpt
        ld = [
            pltpu.async_copy(zeros_hbm.at[pl.ds(r0, rpt)],
                             agg_sh.at[pl.ds(r0, rpt)], gsem.at[0]),
            pltpu.async_copy(degp_hbm.at[0, pl.ds(r0, rpt)], p0, gsem.at[1]),
            pltpu.async_copy(degp_hbm.at[1, pl.ds(r0, rpt)], p1, gsem.at[2]),
            pltpu.async_copy(h1_hbm.at[pl.ds(r0, rpt)], hbuf, gsem.at[3]),
            pltpu.async_copy(src_hbm.at[wid], src_v, gsem.at[4]),
            pltpu.async_copy(dst_hbm.at[wid], dst_v, gsem.at[5]),
        ]
        for d in ld[1:4]:
            d.wait()

        @pl.loop(0, rpt)
        def _(i):
            deg = p0[i, :] + p1[i, :] + 1.0
            dv = _qrsqrt(deg)
            dinvb[i, :] = dv
            zsb[i, :] = hbuf[i, :] * dv

        pltpu.sync_copy(zsb, zs_sh.at[pl.ds(r0, rpt)])

        @pl.when(cid == 0)
        def _():
            pltpu.sync_copy(dinvb, out_hbm.at[2, pl.ds(r0, rpt)])

        ld[0].wait()
        ld[4].wait()
        ld[5].wait()
        plsc.subcore_barrier()
        _agg_pipeline(zs_sh, agg_sh, src_v, dst_v, rows_v, gsem, ssem, cpt)
        plsc.subcore_barrier()
        pltpu.sync_copy(agg_sh.at[pl.ds(r0, rpt)], zsb)

        @pl.loop(0, rpt)
        def _(i):
            zsb[i, :] = zsb[i, :] * dinvb[i, :]

        pltpu.sync_copy(zsb, out_hbm.at[cid, pl.ds(r0, rpt)])

    return pl.kernel(
        body,
        out_type=jax.ShapeDtypeStruct((3, npad, dhid), jnp.float32),
        mesh=_sc_mesh(),
        compiler_params=pltpu.CompilerParams(use_tc_tiling_on_sc=False, needs_layout_passes=False),
        scratch_types=[
            pltpu.VMEM((cpt, CHUNK), jnp.int32),
            pltpu.VMEM((cpt, CHUNK), jnp.int32),
            pltpu.VMEM((NBUF, CHUNK, dhid), jnp.float32),
            pltpu.VMEM((npad // NS, dhid), jnp.float32),
            pltpu.VMEM((npad // NS, dhid), jnp.float32),
            pltpu.VMEM((npad // NS, dhid), jnp.float32),
            pltpu.VMEM((npad // NS, dhid), jnp.float32),
            pltpu.VMEM((npad // NS, dhid), jnp.float32),
            pltpu.SemaphoreType.DMA((NBUF,)),
            pltpu.SemaphoreType.DMA((NBUF,)),
            pltpu.VMEM_SHARED((npad, dhid), jnp.float32),
            pltpu.VMEM_SHARED((npad, dhid), jnp.float32),
        ],
    )


def _make_agg2_kernel(npad, cpt, dhid):
    rpt = npad // NS

    def body(s2_hbm, h1_hbm, b1_hbm, src_hbm, dst_hbm, zeros_hbm, out_hbm,
             src_v, dst_v, rows_v, p0, p1, hbuf, dinvb, zsb, selfb, bbuf,
             gsem, ssem, zs_sh, agg_sh):
        cid = lax.axis_index("c")
        sid = lax.axis_index("s")
        wid = cid * NS + sid
        r0 = sid * rpt
        ld = [
            pltpu.async_copy(zeros_hbm.at[pl.ds(r0, rpt)],
                             agg_sh.at[pl.ds(r0, rpt)], gsem.at[0]),
            pltpu.async_copy(s2_hbm.at[0, pl.ds(r0, rpt)], p0, gsem.at[1]),
            pltpu.async_copy(s2_hbm.at[1, pl.ds(r0, rpt)], p1, gsem.at[2]),
            pltpu.async_copy(s2_hbm.at[2, pl.ds(r0, rpt)], dinvb, gsem.at[3]),
            pltpu.async_copy(h1_hbm.at[pl.ds(r0, rpt)], hbuf, gsem.at[4]),
            pltpu.async_copy(b1_hbm, bbuf, gsem.at[5]),
            pltpu.async_copy(src_hbm.at[wid], src_v, gsem.at[6]),
            pltpu.async_copy(dst_hbm.at[wid], dst_v, gsem.at[7]),
        ]
        for d in ld[1:6]:
            d.wait()

        @pl.loop(0, rpt)
        def _(i):
            dv = dinvb[i, :]
            z = p0[i, :] + p1[i, :] + hbuf[i, :] * dv * dv + bbuf[:]
            z = jnp.maximum(z, 0.0)
            zsb[i, :] = z * dv
            selfb[i, :] = z * dv * dv

        pltpu.sync_copy(zsb, zs_sh.at[pl.ds(r0, rpt)])

        @pl.when(cid == 0)
        def _():
            pltpu.sync_copy(selfb, out_hbm.at[2, pl.ds(r0, rpt)])

        ld[0].wait()
        ld[6].wait()
        ld[7].wait()
        plsc.subcore_barrier()
        _agg_pipeline(zs_sh, agg_sh, src_v, dst_v, rows_v, gsem, ssem, cpt)
        plsc.subcore_barrier()
        pltpu.sync_copy(agg_sh.at[pl.ds(r0, rpt)], zsb)

        @pl.loop(0, rpt)
        def _(i):
            zsb[i, :] = zsb[i, :] * dinvb[i, :]

        pltpu.sync_copy(zsb, out_hbm.at[cid, pl.ds(r0, rpt)])

    return pl.kernel(
        body,
        out_type=jax.ShapeDtypeStruct((3, npad, dhid), jnp.float32),
        mesh=_sc_mesh(),
        compiler_params=pltpu.CompilerParams(use_tc_tiling_on_sc=False, needs_layout_passes=False),
        scratch_types=[
            pltpu.VMEM((cpt, CHUNK), jnp.int32),
            pltpu.VMEM((cpt, CHUNK), jnp.int32),
            pltpu.VMEM((NBUF, CHUNK, dhid), jnp.float32),
            pltpu.VMEM((npad // NS, dhid), jnp.float32),
            pltpu.VMEM((npad // NS, dhid), jnp.float32),
            pltpu.VMEM((npad // NS, dhid), jnp.float32),
            pltpu.VMEM((npad // NS, dhid), jnp.float32),
            pltpu.VMEM((npad // NS, dhid), jnp.float32),
            pltpu.VMEM((npad // NS, dhid), jnp.float32),
            pltpu.VMEM((dhid,), jnp.float32),
            pltpu.SemaphoreType.DMA((NBUF,)),
            pltpu.SemaphoreType.DMA((NBUF,)),
            pltpu.VMEM_SHARED((npad, dhid), jnp.float32),
            pltpu.VMEM_SHARED((npad, dhid), jnp.float32),
        ],
    )


def _m1(x_ref, w_ref, h_ref):
    h_ref[...] = jnp.dot(x_ref[...], w_ref[...],
                         preferred_element_type=jnp.float32)


def _m3(s_ref, w_ref, b_ref, out_ref):
    pre = s_ref[0] + s_ref[1] + s_ref[2]
    out_ref[...] = (
        jnp.dot(pre, w_ref[...], preferred_element_type=jnp.float32) + b_ref[...]
    )


def kernel(x, edge_index, W1, b1, W2, b2):
    n, d_in = x.shape
    d_hid = W1.shape[1]
    e = edge_index.shape[1]

    cpt = -(-e // (NW * CHUNK))
    e_pad = NW * cpt * CHUNK
    npad = -(-(n + 1) // (NS * 8)) * (NS * 8)

    ep = jnp.pad(edge_index, ((0, 0), (0, e_pad - e)), constant_values=n)
    srcp = ep[0].reshape(NW, cpt, CHUNK)
    dstp = ep[1].reshape(NW, cpt, CHUNK)
    zeros_tab = jnp.zeros((npad, d_hid), jnp.float32)
    ones_blk = jnp.ones((CHUNK, d_hid), jnp.float32)

    degp = _make_deg_kernel(npad, cpt, d_hid)(dstp, zeros_tab, ones_blk)

    R = 1000
    G = n // R
    h1 = pl.pallas_call(
        _m1,
        grid=(G,),
        in_specs=[
            pl.BlockSpec((R, d_in), lambda i: (i, 0)),
            pl.BlockSpec((d_in, d_hid), lambda i: (0, 0)),
        ],
        out_specs=pl.BlockSpec((R, d_hid), lambda i: (i, 0)),
        out_shape=jax.ShapeDtypeStruct((npad, d_hid), jnp.float32),
    )(x, W1)

    s2 = _make_agg1_kernel(npad, cpt, d_hid)(degp, h1, srcp, dstp, zeros_tab)
    s3 = _make_agg2_kernel(npad, cpt, d_hid)(s2, h1, b1, srcp, dstp, zeros_tab)

    out = pl.pallas_call(
        _m3,
        grid=(G,),
        in_specs=[
            pl.BlockSpec((3, R, d_hid), lambda i: (0, i, 0)),
            pl.BlockSpec((d_hid, d_in), lambda i: (0, 0)),
            pl.BlockSpec((1, d_in), lambda i: (0, 0)),
        ],
        out_specs=pl.BlockSpec((R, d_in), lambda i: (i, 0)),
        out_shape=jax.ShapeDtypeStruct((n, d_in), jnp.float32),
    )(s3, W2, b2.reshape(1, d_in))

    return out
